# baseline (speedup 1.0000x reference)
import jax
import jax.numpy as jnp
from jax import lax
from jax.experimental import pallas as pl
from jax.experimental.pallas import tpu as pltpu
from jax.experimental.pallas import tpu_sc as plsc

BATCH = 16384
GMF_DIM = 16
MLP_DIM = 32
H1 = 64
H2 = 32
LANES = 128

_NC = 2
_NS = 16
_NW = _NC * _NS
_BPW = BATCH // _NW
_GRP = 16
_NGRP = _BPW // _GRP


def _fire(table, u, slab, slot, dim, sem, sid):
    aligned = pl.multiple_of((u >> 7) * LANES, LANES)
    return pltpu.async_copy(
        table.at[:, pl.ds(aligned, LANES)],
        slab.at[sid, slot, pl.ds(0, dim)], sem.at[slot])


def _drain(table, slab, slot, dim, sem, sid):
    pltpu.make_async_copy(
        table.at[:, pl.ds(0, LANES)],
        slab.at[sid, slot, pl.ds(0, dim)], sem.at[slot]).wait()


def _extract(u, slab, slot, dim, b_out, k):
    iota = lax.iota(jnp.int32, 16)
    lane = jnp.full((16,), u & (LANES - 1), jnp.int32)
    col = jnp.full((16,), k, jnp.int32)
    for half in range(0):
        d_vec = iota + half * 16
        val = plsc.load_gather(slab.at[slot], [d_vec, lane])
        plsc.store_scatter(b_out, [d_vec, col], val)


def _gather_pass(idx_v, table, slab, dim, b_out, sem, sid):
    uvec0 = idx_v[pl.ds(0, _GRP)]
    for j in range(_GRP):
        _fire(table, uvec0[j], slab, j, dim, sem, sid)

    def body(g, carry):
        uvec_prev = idx_v[pl.ds((g - 1) * _GRP, _GRP)]
        uvec = idx_v[pl.ds(g * _GRP, _GRP)]
        for j in range(_GRP):
            _drain(table, slab, j, dim, sem, sid)
            _extract(uvec_prev[j], slab, j, dim, b_out, (g - 1) * _GRP + j)
            _fire(table, uvec[j], slab, j, dim, sem, sid)
        return carry

    lax.fori_loop(1, _NGRP, body, 0)
    uvec_last = idx_v[pl.ds((_NGRP - 1) * _GRP, _GRP)]
    for j in range(_GRP):
        _drain(table, slab, j, dim, sem, sid)
        _extract(uvec_last[j], slab, j, dim, b_out, (_NGRP - 1) * _GRP + j)


def _gather_body(user_hbm, item_hbm, t_gu, t_gi, t_mu, t_mi,
                 o_gu, o_gi, o_mu, o_mi,
                 uidx, iidx, slab, b_gu, b_gi, b_mu, b_mi, sem):
    wid = lax.axis_index("s") * _NC + lax.axis_index("c")
    base = wid * _BPW
    pltpu.sync_copy(user_hbm.at[pl.ds(base, _BPW)], uidx)
    pltpu.sync_copy(item_hbm.at[pl.ds(base, _BPW)], iidx)
    sid = lax.axis_index("s")
    _gather_pass(uidx, t_gu, slab, GMF_DIM, b_gu, sem, sid)
    _gather_pass(iidx, t_gi, slab, GMF_DIM, b_gi, sem, sid)
    _gather_pass(uidx, t_mu, slab, MLP_DIM, b_mu, sem, sid)
    _gather_pass(iidx, t_mi, slab, MLP_DIM, b_mi, sem, sid)
    pltpu.sync_copy(b_gu, o_gu.at[:, pl.ds(base, _BPW)])
    pltpu.sync_copy(b_gi, o_gi.at[:, pl.ds(base, _BPW)])
    pltpu.sync_copy(b_mu, o_mu.at[:, pl.ds(base, _BPW)])
    pltpu.sync_copy(b_mi, o_mi.at[:, pl.ds(base, _BPW)])


def _make_gather():
    mesh = plsc.VectorSubcoreMesh(core_axis_name="c", subcore_axis_name="s")
    return pl.kernel(
        _gather_body,
        mesh=mesh,
        compiler_params=pltpu.CompilerParams(needs_layout_passes=False),
        out_type=[
            jax.ShapeDtypeStruct((GMF_DIM, BATCH), jnp.float32),
            jax.ShapeDtypeStruct((GMF_DIM, BATCH), jnp.float32),
            jax.ShapeDtypeStruct((MLP_DIM, BATCH), jnp.float32),
            jax.ShapeDtypeStruct((MLP_DIM, BATCH), jnp.float32),
        ],
        scratch_types=[
            pltpu.VMEM((_BPW,), jnp.int32),
            pltpu.VMEM((_BPW,), jnp.int32),
            pltpu.VMEM_SHARED((_NS, _GRP, MLP_DIM, LANES), jnp.float32),
            pltpu.VMEM((GMF_DIM, _BPW), jnp.float32),
            pltpu.VMEM((GMF_DIM, _BPW), jnp.float32),
            pltpu.VMEM((MLP_DIM, _BPW), jnp.float32),
            pltpu.VMEM((MLP_DIM, _BPW), jnp.float32),
            pltpu.SemaphoreType.DMA((_GRP,)),
        ],
    )


def _mlp_body(guT, giT, muT, miT, w1aT, w1bT, b1c, w2T, b2c, wog, woh, bo,
              out):
    h1 = jnp.dot(w1aT[...], muT[...], preferred_element_type=jnp.float32)
    h1 = h1 + jnp.dot(w1bT[...], miT[...], preferred_element_type=jnp.float32)
    h1 = jnp.maximum(h1 + b1c[...], 0.0)
    h2 = jnp.dot(w2T[...], h1, preferred_element_type=jnp.float32)
    h2 = jnp.maximum(h2 + b2c[...], 0.0)
    gmf = guT[...] * giT[...]
    logit = (jnp.dot(wog[...], gmf, preferred_element_type=jnp.float32)
             + jnp.dot(woh[...], h2, preferred_element_type=jnp.float32)
             + bo[...])
    out[...] = 1.0 / (1.0 + jnp.exp(-logit))


_BLK = 2048


def _run_mlp(guT, giT, muT, miT, w1aT, w1bT, b1c, w2T, b2c, wog, woh, bo):
    n_blocks = BATCH // _BLK
    full = lambda shape: pl.BlockSpec(shape, lambda i: (0, 0))
    return pl.pallas_call(
        _mlp_body,
        grid=(n_blocks,),
        in_specs=[
            pl.BlockSpec((GMF_DIM, _BLK), lambda i: (0, i)),
            pl.BlockSpec((GMF_DIM, _BLK), lambda i: (0, i)),
            pl.BlockSpec((MLP_DIM, _BLK), lambda i: (0, i)),
            pl.BlockSpec((MLP_DIM, _BLK), lambda i: (0, i)),
            full((H1, MLP_DIM)),
            full((H1, MLP_DIM)),
            full((H1, 1)),
            full((H2, H1)),
            full((H2, 1)),
            full((1, GMF_DIM)),
            full((1, H2)),
            full((1, 1)),
        ],
        out_specs=pl.BlockSpec((1, _BLK), lambda i: (0, i)),
        out_shape=jax.ShapeDtypeStruct((1, BATCH), jnp.float32),
    )(guT, giT, muT, miT, w1aT, w1bT, b1c, w2T, b2c, wog, woh, bo)


def kernel(user, item, gmf_user_w, gmf_item_w, mlp_user_w, mlp_item_w,
           W1, b1, W2, b2, Wo, bo):
    user = user.astype(jnp.int32)
    item = item.astype(jnp.int32)
    t_gu = gmf_user_w.T
    t_gi = gmf_item_w.T
    t_mu = mlp_user_w.T
    t_mi = mlp_item_w.T
    guT, giT, muT, miT = _make_gather()(user, item, t_gu, t_gi, t_mu, t_mi)
    w1aT = W1[:MLP_DIM].T
    w1bT = W1[MLP_DIM:].T
    w2T = W2.T
    wog = Wo[:GMF_DIM, 0].reshape(1, GMF_DIM)
    woh = Wo[GMF_DIM:, 0].reshape(1, H2)
    out = _run_mlp(guT, giT, muT, miT, w1aT, w1bT, b1.reshape(H1, 1), w2T,
                   b2.reshape(H2, 1), wog, woh, bo.reshape(1, 1))
    return out.reshape(BATCH)


# speedup vs baseline: 1.2940x; 1.2940x over previous
import jax
import jax.numpy as jnp
from jax import lax
from jax.experimental import pallas as pl
from jax.experimental.pallas import tpu as pltpu
from jax.experimental.pallas import tpu_sc as plsc

BATCH = 16384
GMF_DIM = 16
MLP_DIM = 32
H1 = 64
H2 = 32
LANES = 128

_NC = 2
_NS = 16
_NW = _NC * _NS
_BPW = BATCH // _NW
_GRP = 16
_NGRP = _BPW // _GRP


def _fire(table, u, slab, slot, dim, sem):
    aligned = pl.multiple_of((u >> 7) * LANES, LANES)
    return pltpu.async_copy(
        table.at[:, pl.ds(aligned, LANES)],
        slab.at[slot, pl.ds(0, dim)], sem.at[slot])


def _drain(table, slab, slot, dim, sem):
    pltpu.make_async_copy(
        table.at[:, pl.ds(0, LANES)],
        slab.at[slot, pl.ds(0, dim)], sem.at[slot]).wait()


def _extract(u, slab, slot, dim, b_out, k):
    iota = lax.iota(jnp.int32, 16)
    lane = jnp.full((16,), u & (LANES - 1), jnp.int32)
    col = jnp.full((16,), k, jnp.int32)
    for half in range(dim // 16):
        d_vec = iota + half * 16
        val = plsc.load_gather(slab.at[slot], [d_vec, lane])
        plsc.store_scatter(b_out, [d_vec, col], val)


def _gather_pass(idx_v, table, slab, dim, b_out, sem):
    uvec0 = idx_v[pl.ds(0, _GRP)]
    for j in range(_GRP):
        _fire(table, uvec0[j], slab, j, dim, sem)

    def body(g, carry):
        uvec_prev = idx_v[pl.ds((g - 1) * _GRP, _GRP)]
        uvec = idx_v[pl.ds(g * _GRP, _GRP)]
        for j in range(_GRP):
            _drain(table, slab, j, dim, sem)
            _extract(uvec_prev[j], slab, j, dim, b_out, (g - 1) * _GRP + j)
            _fire(table, uvec[j], slab, j, dim, sem)
        return carry

    lax.fori_loop(1, _NGRP, body, 0)
    uvec_last = idx_v[pl.ds((_NGRP - 1) * _GRP, _GRP)]
    for j in range(_GRP):
        _drain(table, slab, j, dim, sem)
        _extract(uvec_last[j], slab, j, dim, b_out, (_NGRP - 1) * _GRP + j)


def _gather_body(user_hbm, item_hbm, t_gu, t_gi, t_mu, t_mi,
                 o_gu, o_gi, o_mu, o_mi,
                 uidx, iidx, slab, b_gu, b_gi, b_mu, b_mi, sem):
    wid = lax.axis_index("s") * _NC + lax.axis_index("c")
    base = wid * _BPW
    pltpu.sync_copy(user_hbm.at[pl.ds(base, _BPW)], uidx)
    pltpu.sync_copy(item_hbm.at[pl.ds(base, _BPW)], iidx)
    _gather_pass(uidx, t_gu, slab, GMF_DIM, b_gu, sem)
    _gather_pass(iidx, t_gi, slab, GMF_DIM, b_gi, sem)
    _gather_pass(uidx, t_mu, slab, MLP_DIM, b_mu, sem)
    _gather_pass(iidx, t_mi, slab, MLP_DIM, b_mi, sem)
    pltpu.sync_copy(b_gu, o_gu.at[:, pl.ds(base, _BPW)])
    pltpu.sync_copy(b_gi, o_gi.at[:, pl.ds(base, _BPW)])
    pltpu.sync_copy(b_mu, o_mu.at[:, pl.ds(base, _BPW)])
    pltpu.sync_copy(b_mi, o_mi.at[:, pl.ds(base, _BPW)])


def _make_gather():
    mesh = plsc.VectorSubcoreMesh(core_axis_name="c", subcore_axis_name="s")
    return pl.kernel(
        _gather_body,
        mesh=mesh,
        compiler_params=pltpu.CompilerParams(needs_layout_passes=False),
        out_type=[
            jax.ShapeDtypeStruct((GMF_DIM, BATCH), jnp.float32),
            jax.ShapeDtypeStruct((GMF_DIM, BATCH), jnp.float32),
            jax.ShapeDtypeStruct((MLP_DIM, BATCH), jnp.float32),
            jax.ShapeDtypeStruct((MLP_DIM, BATCH), jnp.float32),
        ],
        scratch_types=[
            pltpu.VMEM((_BPW,), jnp.int32),
            pltpu.VMEM((_BPW,), jnp.int32),
            pltpu.VMEM((_GRP, MLP_DIM, LANES), jnp.float32),
            pltpu.VMEM((GMF_DIM, _BPW), jnp.float32),
            pltpu.VMEM((GMF_DIM, _BPW), jnp.float32),
            pltpu.VMEM((MLP_DIM, _BPW), jnp.float32),
            pltpu.VMEM((MLP_DIM, _BPW), jnp.float32),
            pltpu.SemaphoreType.DMA((_GRP,)),
        ],
    )


def _mlp_body(guT, giT, muT, miT, w1aT, w1bT, b1c, w2T, b2c, wog, woh, bo,
              out):
    h1 = jnp.dot(w1aT[...], muT[...], preferred_element_type=jnp.float32)
    h1 = h1 + jnp.dot(w1bT[...], miT[...], preferred_element_type=jnp.float32)
    h1 = jnp.maximum(h1 + b1c[...], 0.0)
    h2 = jnp.dot(w2T[...], h1, preferred_element_type=jnp.float32)
    h2 = jnp.maximum(h2 + b2c[...], 0.0)
    gmf = guT[...] * giT[...]
    logit = (jnp.dot(wog[...], gmf, preferred_element_type=jnp.float32)
             + jnp.dot(woh[...], h2, preferred_element_type=jnp.float32)
             + bo[...])
    out[...] = 1.0 / (1.0 + jnp.exp(-logit))


_BLK = 2048


def _run_mlp(guT, giT, muT, miT, w1aT, w1bT, b1c, w2T, b2c, wog, woh, bo):
    n_blocks = BATCH // _BLK
    full = lambda shape: pl.BlockSpec(shape, lambda i: (0, 0))
    return pl.pallas_call(
        _mlp_body,
        grid=(n_blocks,),
        in_specs=[
            pl.BlockSpec((GMF_DIM, _BLK), lambda i: (0, i)),
            pl.BlockSpec((GMF_DIM, _BLK), lambda i: (0, i)),
            pl.BlockSpec((MLP_DIM, _BLK), lambda i: (0, i)),
            pl.BlockSpec((MLP_DIM, _BLK), lambda i: (0, i)),
            full((H1, MLP_DIM)),
            full((H1, MLP_DIM)),
            full((H1, 1)),
            full((H2, H1)),
            full((H2, 1)),
            full((1, GMF_DIM)),
            full((1, H2)),
            full((1, 1)),
        ],
        out_specs=pl.BlockSpec((1, _BLK), lambda i: (0, i)),
        out_shape=jax.ShapeDtypeStruct((1, BATCH), jnp.float32),
    )(guT, giT, muT, miT, w1aT, w1bT, b1c, w2T, b2c, wog, woh, bo)


def kernel(user, item, gmf_user_w, gmf_item_w, mlp_user_w, mlp_item_w,
           W1, b1, W2, b2, Wo, bo):
    user = user.astype(jnp.int32)
    item = item.astype(jnp.int32)
    t_gu = gmf_user_w.T
    t_gi = gmf_item_w.T
    t_mu = mlp_user_w.T
    t_mi = mlp_item_w.T
    guT, giT, muT, miT = _make_gather()(user, item, t_gu, t_gi, t_mu, t_mi)
    w1aT = W1[:MLP_DIM].T
    w1bT = W1[MLP_DIM:].T
    w2T = W2.T
    wog = Wo[:GMF_DIM, 0].reshape(1, GMF_DIM)
    woh = Wo[GMF_DIM:, 0].reshape(1, H2)
    out = _run_mlp(guT, giT, muT, miT, w1aT, w1bT, b1.reshape(H1, 1), w2T,
                   b2.reshape(H2, 1), wog, woh, bo.reshape(1, 1))
    return out.reshape(BATCH)
